# baseline (speedup 1.0000x reference)
import functools

import jax
import jax.numpy as jnp
from jax import lax
from jax.experimental import pallas as pl
from jax.experimental.pallas import tpu as pltpu
from jax.experimental.pallas import tpu_sc as plsc

V = 1000
H = 4
D = 1000

_NC = 2
_NS = 16
_NW = _NC * _NS


def _table_kernel(emb_ref, w_ref, b_ref, t_ref):
    t_ref[...] = (
        jnp.dot(emb_ref[...], w_ref[...], preferred_element_type=jnp.float32)
        + b_ref[...]
    )


def _make_gather(Bt, Lt):
    items_per_w = Bt // _NW
    stage = -(-V // _NS)
    stage_last = V - stage * (_NS - 1)
    mesh = plsc.VectorSubcoreMesh(core_axis_name="c", subcore_axis_name="s")

    def _body(table_hbm, idx_hbm, out_hbm, idx_v, rows_v, tbl_sh, gsem0, gsem1):
        cid = lax.axis_index("c")
        sid = lax.axis_index("s")
        wid = sid * _NC + cid
        item0 = wid * items_per_w
        per_w_rows = items_per_w * Lt

        row0 = sid * stage

        @pl.when(sid < _NS - 1)
        def _():
            pltpu.sync_copy(
                table_hbm.at[pl.ds(row0, stage)], tbl_sh.at[pl.ds(row0, stage)]
            )

        @pl.when(sid == _NS - 1)
        def _():
            pltpu.sync_copy(
                table_hbm.at[pl.ds(stage * (_NS - 1), stage_last)],
                tbl_sh.at[pl.ds(stage * (_NS - 1), stage_last)],
            )

        pltpu.sync_copy(idx_hbm.at[pl.ds(item0, items_per_w)], idx_v)
        plsc.subcore_barrier()

        def start_gather(k, buf, sem):
            pltpu.async_copy(
                tbl_sh.at[idx_v.at[k]],
                rows_v.at[buf],
                sem,
            )

        def wait_gather(buf, sem):
            pltpu.make_async_copy(
                table_hbm.at[pl.ds(0, Lt)], rows_v.at[buf], sem
            ).wait()

        def scatter(k, buf):
            pltpu.sync_copy(rows_v.at[buf], out_hbm.at[item0 + k])

        start_gather(0, 0, gsem0)

        def body(i, carry):
            k0 = 2 * i
            start_gather(k0 + 1, 1, gsem1)
            wait_gather(0, gsem0)
            scatter(k0, 0)
            start_gather(jnp.minimum(k0 + 2, items_per_w - 1), 0, gsem0)
            wait_gather(1, gsem1)
            scatter(k0 + 1, 1)
            return carry

        lax.fori_loop(0, items_per_w // 2, body, 0)
        wait_gather(0, gsem0)

    @functools.partial(
        pl.kernel,
        mesh=mesh,
        compiler_params=pltpu.CompilerParams(use_tc_tiling_on_sc=False),
        out_type=jax.ShapeDtypeStruct((Bt, Lt, D), jnp.float32),
        scratch_types=[
            pltpu.VMEM((Bt // _NW, Lt), jnp.int32),
            pltpu.VMEM((2, Lt, D), jnp.float32),
            pltpu.VMEM_SHARED((V, D), jnp.float32),
            pltpu.SemaphoreType.DMA,
            pltpu.SemaphoreType.DMA,
        ],
    )
    def gather(table_hbm, idx_hbm, out_hbm, idx_v, rows_v, tbl_sh, gsem0, gsem1):
        _body(table_hbm, idx_hbm, out_hbm, idx_v, rows_v, tbl_sh, gsem0, gsem1)

    return gather


def kernel(input_ids, emb, W, b):
    Bt, Lt = input_ids.shape
    table = pl.pallas_call(
        _table_kernel,
        out_shape=jax.ShapeDtypeStruct((V, D), jnp.float32),
    )(emb, W, b.reshape(1, V))

    ids = input_ids.astype(jnp.int32)
    return _make_gather(Bt, Lt)(table, ids)
